# initial kernel scaffold (unmeasured)
import jax
import jax.numpy as jnp
from jax import lax
from jax.experimental import pallas as pl
from jax.experimental.pallas import tpu as pltpu


def kernel(x, pi):
    def body(x_ref, pi_ref, out_ref, send_sem, recv_sem, copy_sem):
        my_x = lax.axis_index("x")
        my_y = lax.axis_index("y")
        tgt_y = pi_ref[my_y]

        barrier_sem = pltpu.get_barrier_semaphore()
        pl.semaphore_signal(
            barrier_sem,
            inc=1,
            device_id=(my_x, 1 - my_y),
            device_id_type=pl.DeviceIdType.MESH,
        )
        pl.semaphore_wait(barrier_sem, 1)

        @pl.when(tgt_y != my_y)
        def _():
            rdma = pltpu.make_async_remote_copy(
                src_ref=x_ref,
                dst_ref=out_ref,
                send_sem=send_sem,
                recv_sem=recv_sem,
                device_id=(my_x, tgt_y),
                device_id_type=pl.DeviceIdType.MESH,
            )
            rdma.start()
            rdma.wait()

        @pl.when(tgt_y == my_y)
        def _():
            copy = pltpu.make_async_copy(x_ref, out_ref, copy_sem)
            copy.start()
            copy.wait()

    return pl.pallas_call(
        body,
        out_shape=jax.ShapeDtypeStruct(x.shape, x.dtype),
        in_specs=[
            pl.BlockSpec(memory_space=pltpu.ANY),
            pl.BlockSpec(memory_space=pltpu.SMEM),
        ],
        out_specs=pl.BlockSpec(memory_space=pltpu.ANY),
        scratch_shapes=[
            pltpu.SemaphoreType.DMA,
            pltpu.SemaphoreType.DMA,
            pltpu.SemaphoreType.DMA,
        ],
        compiler_params=pltpu.CompilerParams(collective_id=0),
    )(x, pi)


# baseline (device time: 387346 ns/iter reference)
import jax
import jax.numpy as jnp
from jax import lax
from jax.experimental import pallas as pl
from jax.experimental.pallas import tpu as pltpu


def kernel(x, pi):
    def body(x_ref, pi_ref, out_ref, send_sem, recv_sem, copy_sem):
        my_x = lax.axis_index("x")
        my_y = lax.axis_index("y")
        tgt_y = pi_ref[my_y]

        barrier_sem = pltpu.get_barrier_semaphore()
        pl.semaphore_signal(
            barrier_sem,
            inc=1,
            device_id=(my_x, 1 - my_y),
            device_id_type=pl.DeviceIdType.MESH,
        )
        pl.semaphore_wait(barrier_sem, 1)

        @pl.when(tgt_y != my_y)
        def _():
            rdma = pltpu.make_async_remote_copy(
                src_ref=x_ref,
                dst_ref=out_ref,
                send_sem=send_sem,
                recv_sem=recv_sem,
                device_id=(my_x, tgt_y),
                device_id_type=pl.DeviceIdType.MESH,
            )
            rdma.start()
            rdma.wait()

        @pl.when(tgt_y == my_y)
        def _():
            copy = pltpu.make_async_copy(x_ref, out_ref, copy_sem)
            copy.start()
            copy.wait()

    return pl.pallas_call(
        body,
        out_shape=jax.ShapeDtypeStruct(x.shape, x.dtype),
        in_specs=[
            pl.BlockSpec(memory_space=pl.ANY),
            pl.BlockSpec(memory_space=pltpu.SMEM),
        ],
        out_specs=pl.BlockSpec(memory_space=pl.ANY),
        scratch_shapes=[
            pltpu.SemaphoreType.DMA,
            pltpu.SemaphoreType.DMA,
            pltpu.SemaphoreType.DMA,
        ],
        compiler_params=pltpu.CompilerParams(collective_id=0),
    )(x, pi)


# device time: 233256 ns/iter; 1.6606x vs baseline; 1.6606x over previous
import jax
import jax.numpy as jnp
from jax import lax
from jax.experimental import pallas as pl
from jax.experimental.pallas import tpu as pltpu

C = 8


def kernel(x, pi):
    _, m, n = x.shape
    half = m // 2
    rows = half // C

    def body(x_ref, pi_ref, out_ref, ys_sem, yr_sem, xs_sem, xr_sem, copy_sem):
        my_x = lax.axis_index("x")
        my_y = lax.axis_index("y")
        tgt_y = pi_ref[my_y]

        barrier_sem = pltpu.get_barrier_semaphore()
        pl.semaphore_signal(
            barrier_sem, inc=1,
            device_id=(my_x, 1 - my_y),
            device_id_type=pl.DeviceIdType.MESH,
        )
        pl.semaphore_signal(
            barrier_sem, inc=1,
            device_id=(1 - my_x, my_y),
            device_id_type=pl.DeviceIdType.MESH,
        )
        pl.semaphore_wait(barrier_sem, 2)

        def chunk_start(c):
            return my_x * half + c * rows

        def rdma_y(c):
            s = chunk_start(c)
            return pltpu.make_async_remote_copy(
                src_ref=x_ref.at[0, pl.ds(s, rows), :],
                dst_ref=out_ref.at[0, pl.ds(s, rows), :],
                send_sem=ys_sem.at[c],
                recv_sem=yr_sem.at[c],
                device_id=(my_x, 1 - my_y),
                device_id_type=pl.DeviceIdType.MESH,
            )

        def rdma_x(c):
            s = chunk_start(c)
            return pltpu.make_async_remote_copy(
                src_ref=out_ref.at[0, pl.ds(s, rows), :],
                dst_ref=out_ref.at[0, pl.ds(s, rows), :],
                send_sem=xs_sem.at[c],
                recv_sem=xr_sem.at[c],
                device_id=(1 - my_x, my_y),
                device_id_type=pl.DeviceIdType.MESH,
            )

        @pl.when(tgt_y != my_y)
        def _():
            for c in range(C):
                rdma_y(c).start()
            for c in range(C):
                rdma_y(c).wait_recv()
                rdma_x(c).start()
            for c in range(C):
                rdma_x(c).wait_recv()
            for c in range(C):
                rdma_y(c).wait_send()
                rdma_x(c).wait_send()

        @pl.when(tgt_y == my_y)
        def _():
            copy = pltpu.make_async_copy(x_ref, out_ref, copy_sem)
            copy.start()
            copy.wait()

    return pl.pallas_call(
        body,
        out_shape=jax.ShapeDtypeStruct(x.shape, x.dtype),
        in_specs=[
            pl.BlockSpec(memory_space=pl.ANY),
            pl.BlockSpec(memory_space=pltpu.SMEM),
        ],
        out_specs=pl.BlockSpec(memory_space=pl.ANY),
        scratch_shapes=[
            pltpu.SemaphoreType.DMA((C,)),
            pltpu.SemaphoreType.DMA((C,)),
            pltpu.SemaphoreType.DMA((C,)),
            pltpu.SemaphoreType.DMA((C,)),
            pltpu.SemaphoreType.DMA,
        ],
        compiler_params=pltpu.CompilerParams(collective_id=0),
    )(x, pi)


# device time: 222341 ns/iter; 1.7421x vs baseline; 1.0491x over previous
import jax
import jax.numpy as jnp
from jax import lax
from jax.experimental import pallas as pl
from jax.experimental.pallas import tpu as pltpu

C = 16


def kernel(x, pi):
    _, m, n = x.shape
    half = m // 2
    rows = half // C

    def body(x_ref, pi_ref, out_ref, ys_sem, yr_sem, xs_sem, xr_sem, copy_sem):
        my_x = lax.axis_index("x")
        my_y = lax.axis_index("y")
        tgt_y = pi_ref[my_y]

        barrier_sem = pltpu.get_barrier_semaphore()
        pl.semaphore_signal(
            barrier_sem, inc=1,
            device_id=(my_x, 1 - my_y),
            device_id_type=pl.DeviceIdType.MESH,
        )
        pl.semaphore_signal(
            barrier_sem, inc=1,
            device_id=(1 - my_x, my_y),
            device_id_type=pl.DeviceIdType.MESH,
        )
        pl.semaphore_wait(barrier_sem, 2)

        def chunk_start(c):
            return my_x * half + c * rows

        def rdma_y(c):
            s = chunk_start(c)
            return pltpu.make_async_remote_copy(
                src_ref=x_ref.at[0, pl.ds(s, rows), :],
                dst_ref=out_ref.at[0, pl.ds(s, rows), :],
                send_sem=ys_sem.at[c],
                recv_sem=yr_sem.at[c],
                device_id=(my_x, 1 - my_y),
                device_id_type=pl.DeviceIdType.MESH,
            )

        def rdma_x(c):
            s = chunk_start(c)
            return pltpu.make_async_remote_copy(
                src_ref=out_ref.at[0, pl.ds(s, rows), :],
                dst_ref=out_ref.at[0, pl.ds(s, rows), :],
                send_sem=xs_sem.at[c],
                recv_sem=xr_sem.at[c],
                device_id=(1 - my_x, my_y),
                device_id_type=pl.DeviceIdType.MESH,
            )

        @pl.when(tgt_y != my_y)
        def _():
            for c in range(C):
                rdma_y(c).start()
            for c in range(C):
                rdma_y(c).wait_recv()
                rdma_x(c).start()
            for c in range(C):
                rdma_x(c).wait_recv()
            for c in range(C):
                rdma_y(c).wait_send()
                rdma_x(c).wait_send()

        @pl.when(tgt_y == my_y)
        def _():
            copy = pltpu.make_async_copy(x_ref, out_ref, copy_sem)
            copy.start()
            copy.wait()

    return pl.pallas_call(
        body,
        out_shape=jax.ShapeDtypeStruct(x.shape, x.dtype),
        in_specs=[
            pl.BlockSpec(memory_space=pl.ANY),
            pl.BlockSpec(memory_space=pltpu.SMEM),
        ],
        out_specs=pl.BlockSpec(memory_space=pl.ANY),
        scratch_shapes=[
            pltpu.SemaphoreType.DMA((C,)),
            pltpu.SemaphoreType.DMA((C,)),
            pltpu.SemaphoreType.DMA((C,)),
            pltpu.SemaphoreType.DMA((C,)),
            pltpu.SemaphoreType.DMA,
        ],
        compiler_params=pltpu.CompilerParams(collective_id=0),
    )(x, pi)


# device time: 217453 ns/iter; 1.7813x vs baseline; 1.0225x over previous
import jax
import jax.numpy as jnp
from jax import lax
from jax.experimental import pallas as pl
from jax.experimental.pallas import tpu as pltpu

C = 32


def kernel(x, pi):
    _, m, n = x.shape
    half = m // 2
    rows = half // C

    def body(x_ref, pi_ref, out_ref, ys_sem, yr_sem, xs_sem, xr_sem, copy_sem):
        my_x = lax.axis_index("x")
        my_y = lax.axis_index("y")
        tgt_y = pi_ref[my_y]

        barrier_sem = pltpu.get_barrier_semaphore()
        pl.semaphore_signal(
            barrier_sem, inc=1,
            device_id=(my_x, 1 - my_y),
            device_id_type=pl.DeviceIdType.MESH,
        )
        pl.semaphore_signal(
            barrier_sem, inc=1,
            device_id=(1 - my_x, my_y),
            device_id_type=pl.DeviceIdType.MESH,
        )
        pl.semaphore_wait(barrier_sem, 2)

        def chunk_start(c):
            return my_x * half + c * rows

        def rdma_y(c):
            s = chunk_start(c)
            return pltpu.make_async_remote_copy(
                src_ref=x_ref.at[0, pl.ds(s, rows), :],
                dst_ref=out_ref.at[0, pl.ds(s, rows), :],
                send_sem=ys_sem.at[c],
                recv_sem=yr_sem.at[c],
                device_id=(my_x, 1 - my_y),
                device_id_type=pl.DeviceIdType.MESH,
            )

        def rdma_x(c):
            s = chunk_start(c)
            return pltpu.make_async_remote_copy(
                src_ref=out_ref.at[0, pl.ds(s, rows), :],
                dst_ref=out_ref.at[0, pl.ds(s, rows), :],
                send_sem=xs_sem.at[c],
                recv_sem=xr_sem.at[c],
                device_id=(1 - my_x, my_y),
                device_id_type=pl.DeviceIdType.MESH,
            )

        @pl.when(tgt_y != my_y)
        def _():
            for c in range(C):
                rdma_y(c).start()
            for c in range(C):
                rdma_y(c).wait_recv()
                rdma_x(c).start()
            for c in range(C):
                rdma_x(c).wait_recv()
            for c in range(C):
                rdma_y(c).wait_send()
                rdma_x(c).wait_send()

        @pl.when(tgt_y == my_y)
        def _():
            copy = pltpu.make_async_copy(x_ref, out_ref, copy_sem)
            copy.start()
            copy.wait()

    return pl.pallas_call(
        body,
        out_shape=jax.ShapeDtypeStruct(x.shape, x.dtype),
        in_specs=[
            pl.BlockSpec(memory_space=pl.ANY),
            pl.BlockSpec(memory_space=pltpu.SMEM),
        ],
        out_specs=pl.BlockSpec(memory_space=pl.ANY),
        scratch_shapes=[
            pltpu.SemaphoreType.DMA((C,)),
            pltpu.SemaphoreType.DMA((C,)),
            pltpu.SemaphoreType.DMA((C,)),
            pltpu.SemaphoreType.DMA((C,)),
            pltpu.SemaphoreType.DMA,
        ],
        compiler_params=pltpu.CompilerParams(collective_id=0),
    )(x, pi)


# device time: 216017 ns/iter; 1.7931x vs baseline; 1.0066x over previous
import jax
import jax.numpy as jnp
from jax import lax
from jax.experimental import pallas as pl
from jax.experimental.pallas import tpu as pltpu

C = 64


def kernel(x, pi):
    _, m, n = x.shape
    half = m // 2
    rows = half // C

    def body(x_ref, pi_ref, out_ref, ys_sem, yr_sem, xs_sem, xr_sem, copy_sem):
        my_x = lax.axis_index("x")
        my_y = lax.axis_index("y")
        tgt_y = pi_ref[my_y]

        barrier_sem = pltpu.get_barrier_semaphore()
        pl.semaphore_signal(
            barrier_sem, inc=1,
            device_id=(my_x, 1 - my_y),
            device_id_type=pl.DeviceIdType.MESH,
        )
        pl.semaphore_signal(
            barrier_sem, inc=1,
            device_id=(1 - my_x, my_y),
            device_id_type=pl.DeviceIdType.MESH,
        )
        pl.semaphore_wait(barrier_sem, 2)

        def chunk_start(c):
            return my_x * half + c * rows

        def rdma_y(c):
            s = chunk_start(c)
            return pltpu.make_async_remote_copy(
                src_ref=x_ref.at[0, pl.ds(s, rows), :],
                dst_ref=out_ref.at[0, pl.ds(s, rows), :],
                send_sem=ys_sem.at[c],
                recv_sem=yr_sem.at[c],
                device_id=(my_x, 1 - my_y),
                device_id_type=pl.DeviceIdType.MESH,
            )

        def rdma_x(c):
            s = chunk_start(c)
            return pltpu.make_async_remote_copy(
                src_ref=out_ref.at[0, pl.ds(s, rows), :],
                dst_ref=out_ref.at[0, pl.ds(s, rows), :],
                send_sem=xs_sem.at[c],
                recv_sem=xr_sem.at[c],
                device_id=(1 - my_x, my_y),
                device_id_type=pl.DeviceIdType.MESH,
            )

        @pl.when(tgt_y != my_y)
        def _():
            for c in range(C):
                rdma_y(c).start()
            for c in range(C):
                rdma_y(c).wait_recv()
                rdma_x(c).start()
            for c in range(C):
                rdma_x(c).wait_recv()
            for c in range(C):
                rdma_y(c).wait_send()
                rdma_x(c).wait_send()

        @pl.when(tgt_y == my_y)
        def _():
            copy = pltpu.make_async_copy(x_ref, out_ref, copy_sem)
            copy.start()
            copy.wait()

    return pl.pallas_call(
        body,
        out_shape=jax.ShapeDtypeStruct(x.shape, x.dtype),
        in_specs=[
            pl.BlockSpec(memory_space=pl.ANY),
            pl.BlockSpec(memory_space=pltpu.SMEM),
        ],
        out_specs=pl.BlockSpec(memory_space=pl.ANY),
        scratch_shapes=[
            pltpu.SemaphoreType.DMA((C,)),
            pltpu.SemaphoreType.DMA((C,)),
            pltpu.SemaphoreType.DMA((C,)),
            pltpu.SemaphoreType.DMA((C,)),
            pltpu.SemaphoreType.DMA,
        ],
        compiler_params=pltpu.CompilerParams(collective_id=0),
    )(x, pi)
